# Initial kernel scaffold; baseline (speedup 1.0000x reference)
import jax
import jax.numpy as jnp
from jax.experimental import pallas as pl


def kernel(x, hyperedge_index, W, bias, membrane):
    raise NotImplementedError("write your pallas kernel here")


# speedup vs baseline: 24.3523x; 24.3523x over previous
import functools

import jax
import jax.numpy as jnp
from jax import lax
from jax.experimental import pallas as pl
from jax.experimental.pallas import tpu as pltpu
from jax.experimental.pallas import tpu_sc as plsc

N_NODES = 10000
N_CONN = 320000
D = 128
BETA = 0.9
SPIKE_THRESHOLD = 1.0

NC = 2
NS = 16
NW = NC * NS
CH = 128
CPW = N_CONN // NW
NCH = -(-CPW // CH)
NPAD = NCH * CH - CPW
NROWS = 10112
RPT = NROWS // NS
MMB = NROWS // 8


def _make_sc_pass(with_counts):
    mesh = plsc.VectorSubcoreMesh(core_axis_name="c", subcore_axis_name="s",
                                  num_cores=NC, num_subcores=NS)
    out_type = [jax.ShapeDtypeStruct((NROWS, D), jnp.float32)] * 2
    if with_counts:
        out_type += [jax.ShapeDtypeStruct((NROWS,), jnp.float32)] * 4
    scratch = [pltpu.VMEM_SHARED((NROWS, D), jnp.float32)]
    if with_counts:
        scratch += [pltpu.VMEM_SHARED((NROWS,), jnp.float32)] * 2
    scratch += [
        pltpu.VMEM((NCH, CH), jnp.int32),
        pltpu.VMEM((NCH, CH), jnp.int32),
        pltpu.VMEM((CH, D), jnp.float32),
        pltpu.VMEM((CH,), jnp.float32),
        pltpu.SemaphoreType.DMA,
    ]
    if with_counts:
        scratch += [pltpu.VMEM((NROWS,), jnp.float32)]

    def body(*refs):
        if with_counts:
            (table_hbm, gidx_hbm, sidx_hbm,
             acc0_out, acc1_out, hg0_out, hg1_out, hs0_out, hs1_out,
             acc_sh, hg_sh, hs_sh,
             gidx_v, sidx_v, rows_v, ones_v, sem, hbuf) = refs
        else:
            (table_hbm, gidx_hbm, sidx_hbm,
             acc0_out, acc1_out,
             acc_sh, gidx_v, sidx_v, rows_v, ones_v, sem) = refs
        cid = lax.axis_index("c")
        sid = lax.axis_index("s")
        wid = sid * NC + cid
        base = sid * RPT
        sl = pl.ds(base, RPT)
        pltpu.sync_copy(gidx_hbm.at[wid], gidx_v)
        pltpu.sync_copy(sidx_hbm.at[wid], sidx_v)
        zv = jnp.zeros((16,), jnp.float32)

        def zrow(j, c):
            for i in range(D // 16):
                rows_v[j, pl.ds(i * 16, 16)] = zv
            return c

        lax.fori_loop(0, CH, zrow, 0)
        rem = RPT - 4 * CH
        for k in range(4):
            pltpu.sync_copy(rows_v, acc_sh.at[pl.ds(base + k * CH, CH)])
        pltpu.sync_copy(rows_v.at[pl.ds(0, rem)],
                        acc_sh.at[pl.ds(base + 4 * CH, rem)])
        if with_counts:
            for i in range(CH // 16):
                ones_v[pl.ds(i * 16, 16)] = jnp.full((16,), 1.0, jnp.float32)
            for h_sh in (hg_sh, hs_sh):
                for k in range(4):
                    pltpu.sync_copy(rows_v.at[0],
                                    h_sh.at[pl.ds(base + k * CH, CH)])
                pltpu.sync_copy(rows_v.at[0, pl.ds(0, rem)],
                                h_sh.at[pl.ds(base + 4 * CH, rem)])
        plsc.subcore_barrier()

        def step(j, carry):
            pltpu.async_copy(table_hbm.at[gidx_v.at[j]], rows_v, sem).wait()
            pltpu.sync_copy(rows_v, acc_sh.at[sidx_v.at[j]], add=True)
            if with_counts:
                pltpu.sync_copy(ones_v, hg_sh.at[gidx_v.at[j]], add=True)
                pltpu.sync_copy(ones_v, hs_sh.at[sidx_v.at[j]], add=True)
            return carry

        lax.fori_loop(0, NCH, step, 0)
        plsc.subcore_barrier()
        @pl.when(cid == 0)
        def _():
            pltpu.sync_copy(acc_sh.at[sl], acc0_out.at[sl])

        @pl.when(cid == 1)
        def _():
            pltpu.sync_copy(acc_sh.at[sl], acc1_out.at[sl])

        if with_counts:
            @pl.when((cid == 0) & (sid == 0))
            def _():
                pltpu.sync_copy(hg_sh, hbuf)
                pltpu.sync_copy(hbuf, hg0_out)
                pltpu.sync_copy(hs_sh, hbuf)
                pltpu.sync_copy(hbuf, hs0_out)

            @pl.when((cid == 1) & (sid == 0))
            def _():
                pltpu.sync_copy(hg_sh, hbuf)
                pltpu.sync_copy(hbuf, hg1_out)
                pltpu.sync_copy(hs_sh, hbuf)
                pltpu.sync_copy(hbuf, hs1_out)

    return pl.kernel(body, out_type=tuple(out_type), mesh=mesh,
                     scratch_types=tuple(scratch))


def _mm_body(x_ref, w_ref, o_ref):
    o_ref[...] = lax.dot_general(
        x_ref[...], w_ref[...], (((1,), (1,)), ((), ())),
        preferred_element_type=jnp.float32)


def _scale_body(m0_ref, m1_ref, c0_ref, c1_ref, o_ref):
    cnt = c0_ref[...] + c1_ref[...]
    binv = jnp.where(cnt > 0, 1.0 / cnt, 0.0)
    o_ref[...] = (m0_ref[...] + m1_ref[...]) * binv


def _snn_body(o0_ref, o1_ref, c0_ref, c1_ref, bias_ref, mem_ref, out_ref):
    cnt = c0_ref[...] + c1_ref[...]
    dinv = jnp.where(cnt > 0, 1.0 / cnt, 0.0)
    v = (o0_ref[...] + o1_ref[...]) * dinv + bias_ref[...] + BETA * mem_ref[...]
    out_ref[...] = (v > SPIKE_THRESHOLD).astype(jnp.float32)


def _row_spec():
    return pl.BlockSpec((MMB, D), lambda i: (i, 0))


def _col_spec():
    return pl.BlockSpec((MMB, 1), lambda i: (i, 0))


def kernel(x, hyperedge_index, W, bias, membrane):
    node = hyperedge_index[0]
    edge = hyperedge_index[1]
    fill = (N_NODES + (jnp.arange(NW * NPAD, dtype=jnp.int32)
                       % (NROWS - N_NODES))).reshape(NW, NPAD)
    nodep = jnp.concatenate([node.reshape(NW, CPW), fill], 1).reshape(NW, NCH, CH)
    edgep = jnp.concatenate([edge.reshape(NW, CPW), fill], 1).reshape(NW, NCH, CH)
    x_pad = jnp.pad(x, ((0, NROWS - N_NODES), (0, 0)))

    xw = pl.pallas_call(
        _mm_body,
        grid=(8,),
        in_specs=[_row_spec(), pl.BlockSpec((D, D), lambda i: (0, 0))],
        out_specs=_row_spec(),
        out_shape=jax.ShapeDtypeStruct((NROWS, D), jnp.float32),
    )(x_pad, W)

    m0, m1, hn0, hn1, he0, he1 = _make_sc_pass(True)(xw, nodep, edgep)

    m_scaled = pl.pallas_call(
        _scale_body,
        grid=(8,),
        in_specs=[_row_spec(), _row_spec(), _col_spec(), _col_spec()],
        out_specs=_row_spec(),
        out_shape=jax.ShapeDtypeStruct((NROWS, D), jnp.float32),
    )(m0, m1, he0.reshape(NROWS, 1), he1.reshape(NROWS, 1))

    o0, o1 = _make_sc_pass(False)(m_scaled, edgep, nodep)

    spike = pl.pallas_call(
        _snn_body,
        grid=(8,),
        in_specs=[_row_spec(), _row_spec(), _col_spec(), _col_spec(),
                  pl.BlockSpec((1, D), lambda i: (0, 0)),
                  pl.BlockSpec((1, D), lambda i: (0, 0))],
        out_specs=_row_spec(),
        out_shape=jax.ShapeDtypeStruct((NROWS, D), jnp.float32),
    )(o0, o1, hn0.reshape(NROWS, 1), hn1.reshape(NROWS, 1),
      bias.reshape(1, D), membrane.reshape(1, D))

    return spike[:N_NODES]


# speedup vs baseline: 32.3842x; 1.3298x over previous
import functools

import jax
import jax.numpy as jnp
from jax import lax
from jax.experimental import pallas as pl
from jax.experimental.pallas import tpu as pltpu
from jax.experimental.pallas import tpu_sc as plsc

N_NODES = 10000
N_CONN = 320000
D = 128
BETA = 0.9
SPIKE_THRESHOLD = 1.0

NC = 2
NS = 16
NW = NC * NS
CH = 128
CPW = N_CONN // NW
NCH = 80
NPAD = NCH * CH - CPW
NROWS = 10112
RPT = NROWS // NS
MMB = NROWS // 8


def _make_sc_pass(with_counts):
    mesh = plsc.VectorSubcoreMesh(core_axis_name="c", subcore_axis_name="s",
                                  num_cores=NC, num_subcores=NS)
    out_type = [jax.ShapeDtypeStruct((NROWS, D), jnp.float32)] * 2
    if with_counts:
        out_type += [jax.ShapeDtypeStruct((NROWS,), jnp.float32)] * 4
    scratch = [pltpu.VMEM_SHARED((NROWS, D), jnp.float32)]
    if with_counts:
        scratch += [pltpu.VMEM_SHARED((NROWS,), jnp.float32)] * 2
    scratch += [
        pltpu.VMEM((CH,), jnp.int32),
        pltpu.VMEM((CH,), jnp.int32),
        pltpu.VMEM((CH,), jnp.int32),
        pltpu.VMEM((CH,), jnp.int32),
        pltpu.VMEM((CH, D), jnp.float32),
        pltpu.VMEM((CH, D), jnp.float32),
        pltpu.VMEM((CH,), jnp.float32),
        pltpu.SemaphoreType.DMA,
        pltpu.SemaphoreType.DMA,
        pltpu.SemaphoreType.DMA,
        pltpu.SemaphoreType.DMA,
    ]
    if with_counts:
        scratch += [pltpu.VMEM((NROWS // 8,), jnp.float32)]

    def body(*refs):
        if with_counts:
            (table_hbm, gidx_hbm, sidx_hbm,
             acc0_out, acc1_out, hg0_out, hg1_out, hs0_out, hs1_out,
             acc_sh, hg_sh, hs_sh,
             gb0, sb0, gb1, sb1, rows0_v, rows1_v, ones_v,
             semi0, semi1, semr0, semr1, hbuf) = refs
        else:
            (table_hbm, gidx_hbm, sidx_hbm,
             acc0_out, acc1_out,
             acc_sh,
             gb0, sb0, gb1, sb1, rows0_v, rows1_v, ones_v,
             semi0, semi1, semr0, semr1) = refs
        cid = lax.axis_index("c")
        sid = lax.axis_index("s")
        wid = sid * NC + cid
        base = sid * RPT
        sl = pl.ds(base, RPT)
        zv = jnp.zeros((16,), jnp.float32)

        def zrow(j, c):
            for i in range(D // 16):
                rows0_v[j, pl.ds(i * 16, 16)] = zv
            return c

        lax.fori_loop(0, CH, zrow, 0)
        rem = RPT - 4 * CH
        for k in range(4):
            pltpu.sync_copy(rows0_v, acc_sh.at[pl.ds(base + k * CH, CH)])
        pltpu.sync_copy(rows0_v.at[pl.ds(0, rem)],
                        acc_sh.at[pl.ds(base + 4 * CH, rem)])
        if with_counts:
            for i in range(CH // 16):
                ones_v[pl.ds(i * 16, 16)] = jnp.full((16,), 1.0, jnp.float32)
            for h_sh in (hg_sh, hs_sh):
                for k in range(4):
                    pltpu.sync_copy(rows0_v.at[0],
                                    h_sh.at[pl.ds(base + k * CH, CH)])
                pltpu.sync_copy(rows0_v.at[0, pl.ds(0, rem)],
                                h_sh.at[pl.ds(base + 4 * CH, rem)])
        plsc.subcore_barrier()

        def idxload(j, gb, sb, sem):
            pltpu.async_copy(gidx_hbm.at[wid, j], gb, sem)
            pltpu.async_copy(sidx_hbm.at[wid, j], sb, sem)

        def idxwait(gb, sb, sem):
            pltpu.make_async_copy(gidx_hbm.at[wid, 0], gb, sem).wait()
            pltpu.make_async_copy(sidx_hbm.at[wid, 0], sb, sem).wait()

        def rows_gather(gb, buf, sem):
            pltpu.async_copy(table_hbm.at[gb], buf, sem)

        def rows_wait(gb, buf, sem):
            pltpu.make_async_copy(table_hbm.at[gb], buf, sem).wait()

        def consume(gb, sb, buf):
            pltpu.sync_copy(buf, acc_sh.at[sb], add=True)
            if with_counts:
                pltpu.sync_copy(ones_v, hg_sh.at[gb], add=True)
                pltpu.sync_copy(ones_v, hs_sh.at[sb], add=True)

        idxload(0, gb0, sb0, semi0)
        idxwait(gb0, sb0, semi0)
        rows_gather(gb0, rows0_v, semr0)
        idxload(1, gb1, sb1, semi1)
        idxwait(gb1, sb1, semi1)

        def pair(t, carry):
            j0 = 2 * t
            rows_gather(gb1, rows1_v, semr1)
            rows_wait(gb0, rows0_v, semr0)
            consume(gb0, sb0, rows0_v)

            @pl.when(j0 + 2 < NCH)
            def _():
                idxload(j0 + 2, gb0, sb0, semi0)
                idxwait(gb0, sb0, semi0)
                rows_gather(gb0, rows0_v, semr0)

            rows_wait(gb1, rows1_v, semr1)
            consume(gb1, sb1, rows1_v)

            @pl.when(j0 + 3 < NCH)
            def _():
                idxload(j0 + 3, gb1, sb1, semi1)
                idxwait(gb1, sb1, semi1)
            return carry

        lax.fori_loop(0, NCH // 2, pair, 0)
        plsc.subcore_barrier()
        @pl.when(cid == 0)
        def _():
            pltpu.sync_copy(acc_sh.at[sl], acc0_out.at[sl])

        @pl.when(cid == 1)
        def _():
            pltpu.sync_copy(acc_sh.at[sl], acc1_out.at[sl])

        if with_counts:
            hp = NROWS // 8

            def hist_write(h_sh, h_out):
                for p in range(8):
                    ps = pl.ds(p * hp, hp)
                    pltpu.sync_copy(h_sh.at[ps], hbuf)
                    pltpu.sync_copy(hbuf, h_out.at[ps])

            @pl.when((cid == 0) & (sid == 0))
            def _():
                hist_write(hg_sh, hg0_out)
                hist_write(hs_sh, hs0_out)

            @pl.when((cid == 1) & (sid == 0))
            def _():
                hist_write(hg_sh, hg1_out)
                hist_write(hs_sh, hs1_out)

    return pl.kernel(body, out_type=tuple(out_type), mesh=mesh,
                     scratch_types=tuple(scratch))


def _mm_body(x_ref, w_ref, o_ref):
    o_ref[...] = lax.dot_general(
        x_ref[...], w_ref[...], (((1,), (1,)), ((), ())),
        preferred_element_type=jnp.float32)


def _scale_body(m0_ref, m1_ref, c0_ref, c1_ref, o_ref):
    cnt = c0_ref[...] + c1_ref[...]
    binv = jnp.where(cnt > 0, 1.0 / cnt, 0.0)
    o_ref[...] = (m0_ref[...] + m1_ref[...]) * binv


def _snn_body(o0_ref, o1_ref, c0_ref, c1_ref, bias_ref, mem_ref, out_ref):
    cnt = c0_ref[...] + c1_ref[...]
    dinv = jnp.where(cnt > 0, 1.0 / cnt, 0.0)
    v = (o0_ref[...] + o1_ref[...]) * dinv + bias_ref[...] + BETA * mem_ref[...]
    out_ref[...] = (v > SPIKE_THRESHOLD).astype(jnp.float32)


def _row_spec():
    return pl.BlockSpec((MMB, D), lambda i: (i, 0))


def _col_spec():
    return pl.BlockSpec((MMB, 1), lambda i: (i, 0))


def kernel(x, hyperedge_index, W, bias, membrane):
    node = hyperedge_index[0]
    edge = hyperedge_index[1]
    fill = (N_NODES + (jnp.arange(NW * NPAD, dtype=jnp.int32)
                       % (NROWS - N_NODES))).reshape(NW, NPAD)
    nodep = jnp.concatenate([node.reshape(NW, CPW), fill], 1).reshape(NW, NCH, CH)
    edgep = jnp.concatenate([edge.reshape(NW, CPW), fill], 1).reshape(NW, NCH, CH)
    x_pad = jnp.pad(x, ((0, NROWS - N_NODES), (0, 0)))

    xw = pl.pallas_call(
        _mm_body,
        grid=(8,),
        in_specs=[_row_spec(), pl.BlockSpec((D, D), lambda i: (0, 0))],
        out_specs=_row_spec(),
        out_shape=jax.ShapeDtypeStruct((NROWS, D), jnp.float32),
    )(x_pad, W)

    m0, m1, hn0, hn1, he0, he1 = _make_sc_pass(True)(xw, nodep, edgep)

    m_scaled = pl.pallas_call(
        _scale_body,
        grid=(8,),
        in_specs=[_row_spec(), _row_spec(), _col_spec(), _col_spec()],
        out_specs=_row_spec(),
        out_shape=jax.ShapeDtypeStruct((NROWS, D), jnp.float32),
    )(m0, m1, he0.reshape(NROWS, 1), he1.reshape(NROWS, 1))

    o0, o1 = _make_sc_pass(False)(m_scaled, edgep, nodep)

    spike = pl.pallas_call(
        _snn_body,
        grid=(8,),
        in_specs=[_row_spec(), _row_spec(), _col_spec(), _col_spec(),
                  pl.BlockSpec((1, D), lambda i: (0, 0)),
                  pl.BlockSpec((1, D), lambda i: (0, 0))],
        out_specs=_row_spec(),
        out_shape=jax.ShapeDtypeStruct((NROWS, D), jnp.float32),
    )(o0, o1, hn0.reshape(NROWS, 1), hn1.reshape(NROWS, 1),
      bias.reshape(1, D), membrane.reshape(1, D))

    return spike[:N_NODES]


# speedup vs baseline: 36.3541x; 1.1226x over previous
import functools

import jax
import jax.numpy as jnp
from jax import lax
from jax.experimental import pallas as pl
from jax.experimental.pallas import tpu as pltpu
from jax.experimental.pallas import tpu_sc as plsc

N_NODES = 10000
N_CONN = 320000
D = 128
BETA = 0.9
SPIKE_THRESHOLD = 1.0

NC = 2
NS = 16
NW = NC * NS
CH = 128
NCHT = N_CONN // CH
NCHW = (NCHT // NW) & ~1
NEXTRA = NCHT - NCHW * NW
NROWS = N_NODES
RPT = 632
RPT_LAST = NROWS - 15 * RPT
MMB = NROWS // 10


def _make_sc_pass(with_counts):
    mesh = plsc.VectorSubcoreMesh(core_axis_name="c", subcore_axis_name="s",
                                  num_cores=NC, num_subcores=NS)
    out_type = [jax.ShapeDtypeStruct((NROWS, D), jnp.float32)] * 2
    if with_counts:
        out_type += [jax.ShapeDtypeStruct((NROWS,), jnp.float32)] * 4
    scratch = [pltpu.VMEM_SHARED((NROWS, D), jnp.float32)]
    if with_counts:
        scratch += [pltpu.VMEM_SHARED((NROWS,), jnp.float32)] * 2
    scratch += [
        pltpu.VMEM((CH,), jnp.int32),
        pltpu.VMEM((CH,), jnp.int32),
        pltpu.VMEM((CH,), jnp.int32),
        pltpu.VMEM((CH,), jnp.int32),
        pltpu.VMEM((CH, D), jnp.float32),
        pltpu.VMEM((CH, D), jnp.float32),
        pltpu.VMEM((CH,), jnp.float32),
        pltpu.SemaphoreType.DMA,
        pltpu.SemaphoreType.DMA,
        pltpu.SemaphoreType.DMA,
        pltpu.SemaphoreType.DMA,
    ]
    if with_counts:
        scratch += [pltpu.VMEM((MMB,), jnp.float32)]

    def body(*refs):
        if with_counts:
            (table_hbm, gidx_hbm, sidx_hbm,
             acc0_out, acc1_out, hg0_out, hg1_out, hs0_out, hs1_out,
             acc_sh, hg_sh, hs_sh,
             gb0, sb0, gb1, sb1, rows0_v, rows1_v, ones_v,
             semi0, semi1, semr0, semr1, hbuf) = refs
        else:
            (table_hbm, gidx_hbm, sidx_hbm,
             acc0_out, acc1_out,
             acc_sh,
             gb0, sb0, gb1, sb1, rows0_v, rows1_v, ones_v,
             semi0, semi1, semr0, semr1) = refs
        cid = lax.axis_index("c")
        sid = lax.axis_index("s")
        wid = sid * NC + cid
        base = sid * RPT
        zv = jnp.zeros((16,), jnp.float32)

        def zrow(j, c):
            for i in range(D // 16):
                rows0_v[j, pl.ds(i * 16, 16)] = zv
            return c

        lax.fori_loop(0, CH, zrow, 0)

        def zero_slices(nrows):
            for k in range(nrows // CH):
                pltpu.sync_copy(rows0_v, acc_sh.at[pl.ds(base + k * CH, CH)])
            rem = nrows % CH
            pltpu.sync_copy(rows0_v.at[pl.ds(0, rem)],
                            acc_sh.at[pl.ds(base + nrows - rem, rem)])
            if with_counts:
                for h_sh in (hg_sh, hs_sh):
                    for k in range(nrows // CH):
                        pltpu.sync_copy(rows0_v.at[0],
                                        h_sh.at[pl.ds(base + k * CH, CH)])
                    pltpu.sync_copy(rows0_v.at[0, pl.ds(0, rem)],
                                    h_sh.at[pl.ds(base + nrows - rem, rem)])

        @pl.when(sid < 15)
        def _():
            zero_slices(RPT)

        @pl.when(sid == 15)
        def _():
            zero_slices(RPT_LAST)

        if with_counts:
            for i in range(CH // 16):
                ones_v[pl.ds(i * 16, 16)] = jnp.full((16,), 1.0, jnp.float32)
        plsc.subcore_barrier()

        def idxload(c, gb, sb, sem):
            pltpu.async_copy(gidx_hbm.at[c, 0], gb, sem)
            pltpu.async_copy(sidx_hbm.at[c, 0], sb, sem)

        def idxwait(gb, sb, sem):
            pltpu.make_async_copy(gidx_hbm.at[0, 0], gb, sem).wait()
            pltpu.make_async_copy(sidx_hbm.at[0, 0], sb, sem).wait()

        def rows_gather(gb, buf, sem):
            pltpu.async_copy(table_hbm.at[gb], buf, sem)

        def rows_wait(gb, buf, sem):
            pltpu.make_async_copy(table_hbm.at[gb], buf, sem).wait()

        def consume(gb, sb, buf):
            pltpu.sync_copy(buf, acc_sh.at[sb], add=True)
            if with_counts:
                pltpu.sync_copy(ones_v, hg_sh.at[gb], add=True)
                pltpu.sync_copy(ones_v, hs_sh.at[sb], add=True)

        idxload(wid, gb0, sb0, semi0)
        idxwait(gb0, sb0, semi0)
        rows_gather(gb0, rows0_v, semr0)
        idxload(wid + NW, gb1, sb1, semi1)
        idxwait(gb1, sb1, semi1)

        def pair(t, carry):
            j0 = 2 * t
            c0 = wid + NW * j0
            rows_gather(gb1, rows1_v, semr1)
            rows_wait(gb0, rows0_v, semr0)
            consume(gb0, sb0, rows0_v)

            @pl.when(j0 + 2 < NCHW)
            def _():
                idxload(c0 + 2 * NW, gb0, sb0, semi0)
                idxwait(gb0, sb0, semi0)
                rows_gather(gb0, rows0_v, semr0)

            rows_wait(gb1, rows1_v, semr1)
            consume(gb1, sb1, rows1_v)

            @pl.when(j0 + 3 < NCHW)
            def _():
                idxload(c0 + 3 * NW, gb1, sb1, semi1)
                idxwait(gb1, sb1, semi1)
            return carry

        lax.fori_loop(0, NCHW // 2, pair, 0)

        @pl.when(wid < NEXTRA)
        def _():
            c = NCHW * NW + wid
            idxload(c, gb0, sb0, semi0)
            idxwait(gb0, sb0, semi0)
            rows_gather(gb0, rows0_v, semr0)
            rows_wait(gb0, rows0_v, semr0)
            consume(gb0, sb0, rows0_v)

        plsc.subcore_barrier()

        def writeback(nrows, acc_out):
            sl = pl.ds(base, nrows)
            pltpu.sync_copy(acc_sh.at[sl], acc_out.at[sl])

        @pl.when((cid == 0) & (sid < 15))
        def _():
            writeback(RPT, acc0_out)

        @pl.when((cid == 0) & (sid == 15))
        def _():
            writeback(RPT_LAST, acc0_out)

        @pl.when((cid == 1) & (sid < 15))
        def _():
            writeback(RPT, acc1_out)

        @pl.when((cid == 1) & (sid == 15))
        def _():
            writeback(RPT_LAST, acc1_out)

        if with_counts:
            def hist_write(h_sh, h_out):
                for p in range(NROWS // MMB):
                    ps = pl.ds(p * MMB, MMB)
                    pltpu.sync_copy(h_sh.at[ps], hbuf)
                    pltpu.sync_copy(hbuf, h_out.at[ps])

            @pl.when((cid == 0) & (sid == 0))
            def _():
                hist_write(hg_sh, hg0_out)
                hist_write(hs_sh, hs0_out)

            @pl.when((cid == 1) & (sid == 0))
            def _():
                hist_write(hg_sh, hg1_out)
                hist_write(hs_sh, hs1_out)

    return pl.kernel(body, out_type=tuple(out_type), mesh=mesh,
                     scratch_types=tuple(scratch))


def _mm_body(x_ref, w_ref, o_ref):
    o_ref[...] = lax.dot_general(
        x_ref[...], w_ref[...], (((1,), (1,)), ((), ())),
        preferred_element_type=jnp.float32)


def _inv_cnt(c0_ref, c1_ref):
    cnt = c0_ref[...] + c1_ref[...]
    return jnp.where(cnt > 0, 1.0 / cnt, 0.0)[:, None]


def _scale_body(m0_ref, m1_ref, c0_ref, c1_ref, o_ref):
    o_ref[...] = (m0_ref[...] + m1_ref[...]) * _inv_cnt(c0_ref, c1_ref)


def _snn_body(o0_ref, o1_ref, c0_ref, c1_ref, bias_ref, mem_ref, out_ref):
    v = ((o0_ref[...] + o1_ref[...]) * _inv_cnt(c0_ref, c1_ref)
         + bias_ref[...] + BETA * mem_ref[...])
    out_ref[...] = (v > SPIKE_THRESHOLD).astype(jnp.float32)


def _row_spec():
    return pl.BlockSpec((MMB, D), lambda i: (i, 0))


def kernel(x, hyperedge_index, W, bias, membrane):
    nodep = hyperedge_index[0].reshape(NCHT, 1, CH)
    edgep = hyperedge_index[1].reshape(NCHT, 1, CH)

    xw = pl.pallas_call(
        _mm_body,
        grid=(10,),
        in_specs=[_row_spec(), pl.BlockSpec((D, D), lambda i: (0, 0))],
        out_specs=_row_spec(),
        out_shape=jax.ShapeDtypeStruct((NROWS, D), jnp.float32),
    )(x, W)

    m0, m1, hn0, hn1, he0, he1 = _make_sc_pass(True)(xw, nodep, edgep)

    m_scaled = pl.pallas_call(
        _scale_body,
        out_shape=jax.ShapeDtypeStruct((NROWS, D), jnp.float32),
    )(m0, m1, he0, he1)

    o0, o1 = _make_sc_pass(False)(m_scaled, edgep, nodep)

    spike = pl.pallas_call(
        _snn_body,
        out_shape=jax.ShapeDtypeStruct((NROWS, D), jnp.float32),
    )(o0, o1, hn0, hn1, bias.reshape(1, D), membrane.reshape(1, D))

    return spike


# speedup vs baseline: 36.6016x; 1.0068x over previous
import functools

import jax
import jax.numpy as jnp
from jax import lax
from jax.experimental import pallas as pl
from jax.experimental.pallas import tpu as pltpu
from jax.experimental.pallas import tpu_sc as plsc

N_NODES = 10000
N_CONN = 320000
D = 128
BETA = 0.9
SPIKE_THRESHOLD = 1.0

NC = 2
NS = 16
NW = NC * NS
CH = 128
NCHT = N_CONN // CH
NCHW = (NCHT // NW) & ~1
NEXTRA = NCHT - NCHW * NW
NROWS = N_NODES
RPT = 632
RPT_LAST = NROWS - 15 * RPT
MMB = NROWS // 10


def _make_sc_pass(with_counts):
    mesh = plsc.VectorSubcoreMesh(core_axis_name="c", subcore_axis_name="s",
                                  num_cores=NC, num_subcores=NS)
    out_type = [jax.ShapeDtypeStruct((NROWS, D), jnp.float32)] * 2
    if with_counts:
        out_type += [jax.ShapeDtypeStruct((NROWS,), jnp.float32)] * 4
    scratch = [pltpu.VMEM_SHARED((NROWS, D), jnp.float32)]
    if with_counts:
        scratch += [pltpu.VMEM_SHARED((NROWS,), jnp.float32)] * 2
    scratch += [
        pltpu.VMEM((CH,), jnp.int32),
        pltpu.VMEM((CH,), jnp.int32),
        pltpu.VMEM((CH,), jnp.int32),
        pltpu.VMEM((CH,), jnp.int32),
        pltpu.VMEM((CH, D), jnp.float32),
        pltpu.VMEM((CH, D), jnp.float32),
        pltpu.VMEM((CH,), jnp.float32),
        pltpu.SemaphoreType.DMA,
        pltpu.SemaphoreType.DMA,
        pltpu.SemaphoreType.DMA,
        pltpu.SemaphoreType.DMA,
    ]
    if with_counts:
        scratch += [pltpu.VMEM((MMB,), jnp.float32)]

    def body(*refs):
        if with_counts:
            (table_hbm, gidx_hbm, sidx_hbm,
             acc0_out, acc1_out, hg0_out, hg1_out, hs0_out, hs1_out,
             acc_sh, hg_sh, hs_sh,
             gb0, sb0, gb1, sb1, rows0_v, rows1_v, ones_v,
             semi0, semi1, semr0, semr1, hbuf) = refs
        else:
            (table_hbm, gidx_hbm, sidx_hbm,
             acc0_out, acc1_out,
             acc_sh,
             gb0, sb0, gb1, sb1, rows0_v, rows1_v, ones_v,
             semi0, semi1, semr0, semr1) = refs
        cid = lax.axis_index("c")
        sid = lax.axis_index("s")
        wid = sid * NC + cid
        base = sid * RPT
        zv = jnp.zeros((16,), jnp.float32)

        def zrow(j, c):
            for i in range(D // 16):
                rows0_v[j, pl.ds(i * 16, 16)] = zv
            return c

        lax.fori_loop(0, CH, zrow, 0)

        def zero_slices(nrows):
            for k in range(nrows // CH):
                pltpu.sync_copy(rows0_v, acc_sh.at[pl.ds(base + k * CH, CH)])
            rem = nrows % CH
            pltpu.sync_copy(rows0_v.at[pl.ds(0, rem)],
                            acc_sh.at[pl.ds(base + nrows - rem, rem)])
            if with_counts:
                for h_sh in (hg_sh, hs_sh):
                    for k in range(nrows // CH):
                        pltpu.sync_copy(rows0_v.at[0],
                                        h_sh.at[pl.ds(base + k * CH, CH)])
                    pltpu.sync_copy(rows0_v.at[0, pl.ds(0, rem)],
                                    h_sh.at[pl.ds(base + nrows - rem, rem)])

        @pl.when(sid < 15)
        def _():
            zero_slices(RPT)

        @pl.when(sid == 15)
        def _():
            zero_slices(RPT_LAST)

        if with_counts:
            for i in range(CH // 16):
                ones_v[pl.ds(i * 16, 16)] = jnp.full((16,), 1.0, jnp.float32)
        plsc.subcore_barrier()

        base_c = NCHW * wid + jnp.minimum(wid, NEXTRA)

        def idxload(c, gb, sb, sem):
            pltpu.async_copy(gidx_hbm.at[c, 0], gb, sem)
            pltpu.async_copy(sidx_hbm.at[c, 0], sb, sem)

        def idxwait(gb, sb, sem):
            pltpu.make_async_copy(gidx_hbm.at[0, 0], gb, sem).wait()
            pltpu.make_async_copy(sidx_hbm.at[0, 0], sb, sem).wait()

        def rows_gather(gb, buf, sem):
            pltpu.async_copy(table_hbm.at[gb], buf, sem)

        def rows_wait(gb, buf, sem):
            pltpu.make_async_copy(table_hbm.at[gb], buf, sem).wait()

        def consume(gb, sb, buf):
            pltpu.sync_copy(buf, acc_sh.at[sb], add=True)
            if with_counts:
                pltpu.sync_copy(ones_v, hg_sh.at[gb], add=True)
                pltpu.sync_copy(ones_v, hs_sh.at[sb], add=True)

        idxload(base_c, gb0, sb0, semi0)
        idxwait(gb0, sb0, semi0)
        rows_gather(gb0, rows0_v, semr0)
        idxload(base_c + 1, gb1, sb1, semi1)
        idxwait(gb1, sb1, semi1)

        def pair(t, carry):
            j0 = 2 * t
            c0 = base_c + j0
            rows_gather(gb1, rows1_v, semr1)
            rows_wait(gb0, rows0_v, semr0)
            consume(gb0, sb0, rows0_v)

            @pl.when(j0 + 2 < NCHW)
            def _():
                idxload(c0 + 2, gb0, sb0, semi0)
                idxwait(gb0, sb0, semi0)
                rows_gather(gb0, rows0_v, semr0)

            rows_wait(gb1, rows1_v, semr1)
            consume(gb1, sb1, rows1_v)

            @pl.when(j0 + 3 < NCHW)
            def _():
                idxload(c0 + 3, gb1, sb1, semi1)
                idxwait(gb1, sb1, semi1)
            return carry

        lax.fori_loop(0, NCHW // 2, pair, 0)

        @pl.when(wid < NEXTRA)
        def _():
            c = base_c + NCHW
            idxload(c, gb0, sb0, semi0)
            idxwait(gb0, sb0, semi0)
            rows_gather(gb0, rows0_v, semr0)
            rows_wait(gb0, rows0_v, semr0)
            consume(gb0, sb0, rows0_v)

        plsc.subcore_barrier()

        def writeback(nrows, acc_out):
            sl = pl.ds(base, nrows)
            pltpu.sync_copy(acc_sh.at[sl], acc_out.at[sl])

        @pl.when((cid == 0) & (sid < 15))
        def _():
            writeback(RPT, acc0_out)

        @pl.when((cid == 0) & (sid == 15))
        def _():
            writeback(RPT_LAST, acc0_out)

        @pl.when((cid == 1) & (sid < 15))
        def _():
            writeback(RPT, acc1_out)

        @pl.when((cid == 1) & (sid == 15))
        def _():
            writeback(RPT_LAST, acc1_out)

        if with_counts:
            def hist_write(h_sh, h_out):
                for p in range(NROWS // MMB):
                    ps = pl.ds(p * MMB, MMB)
                    pltpu.sync_copy(h_sh.at[ps], hbuf)
                    pltpu.sync_copy(hbuf, h_out.at[ps])

            @pl.when((cid == 0) & (sid == 0))
            def _():
                hist_write(hg_sh, hg0_out)
                hist_write(hs_sh, hs0_out)

            @pl.when((cid == 1) & (sid == 0))
            def _():
                hist_write(hg_sh, hg1_out)
                hist_write(hs_sh, hs1_out)

    return pl.kernel(body, out_type=tuple(out_type), mesh=mesh,
                     scratch_types=tuple(scratch))


def _mm_body(x_ref, w_ref, o_ref):
    o_ref[...] = lax.dot_general(
        x_ref[...], w_ref[...], (((1,), (1,)), ((), ())),
        preferred_element_type=jnp.float32)


def _inv_cnt(c0_ref, c1_ref):
    cnt = c0_ref[...] + c1_ref[...]
    return jnp.where(cnt > 0, 1.0 / cnt, 0.0)[:, None]


def _scale_body(m0_ref, m1_ref, c0_ref, c1_ref, o_ref):
    o_ref[...] = (m0_ref[...] + m1_ref[...]) * _inv_cnt(c0_ref, c1_ref)


def _snn_body(o0_ref, o1_ref, c0_ref, c1_ref, bias_ref, mem_ref, out_ref):
    v = ((o0_ref[...] + o1_ref[...]) * _inv_cnt(c0_ref, c1_ref)
         + bias_ref[...] + BETA * mem_ref[...])
    out_ref[...] = (v > SPIKE_THRESHOLD).astype(jnp.float32)


def _row_spec():
    return pl.BlockSpec((MMB, D), lambda i: (i, 0))


def kernel(x, hyperedge_index, W, bias, membrane):
    nodep = hyperedge_index[0].reshape(NCHT, 1, CH)
    edgep = hyperedge_index[1].reshape(NCHT, 1, CH)

    xw = pl.pallas_call(
        _mm_body,
        grid=(10,),
        in_specs=[_row_spec(), pl.BlockSpec((D, D), lambda i: (0, 0))],
        out_specs=_row_spec(),
        out_shape=jax.ShapeDtypeStruct((NROWS, D), jnp.float32),
    )(x, W)

    m0, m1, hn0, hn1, he0, he1 = _make_sc_pass(True)(xw, nodep, edgep)

    m_scaled = pl.pallas_call(
        _scale_body,
        out_shape=jax.ShapeDtypeStruct((NROWS, D), jnp.float32),
    )(m0, m1, he0, he1)

    o0, o1 = _make_sc_pass(False)(m_scaled, edgep, nodep)

    spike = pl.pallas_call(
        _snn_body,
        out_shape=jax.ShapeDtypeStruct((NROWS, D), jnp.float32),
    )(o0, o1, hn0, hn1, bias.reshape(1, D), membrane.reshape(1, D))

    return spike


# speedup vs baseline: 37.7600x; 1.0316x over previous
import functools

import jax
import jax.numpy as jnp
from jax import lax
from jax.experimental import pallas as pl
from jax.experimental.pallas import tpu as pltpu
from jax.experimental.pallas import tpu_sc as plsc

N_NODES = 10000
N_CONN = 320000
D = 128
BETA = 0.9
SPIKE_THRESHOLD = 1.0

NC = 2
NS = 16
NW = NC * NS
CH = 128
NCHT = N_CONN // CH
NCHW = (NCHT // NW) & ~1
NEXTRA = NCHT - NCHW * NW
NROWS = N_NODES
RPT = 632
RPT_LAST = NROWS - 15 * RPT
MMB = NROWS // 10


def _make_sc_pass(with_counts):
    mesh = plsc.VectorSubcoreMesh(core_axis_name="c", subcore_axis_name="s",
                                  num_cores=NC, num_subcores=NS)
    out_type = [jax.ShapeDtypeStruct((NROWS, D), jnp.float32)] * 2
    if with_counts:
        out_type += [jax.ShapeDtypeStruct((NROWS,), jnp.float32)] * 4
    scratch = [pltpu.VMEM_SHARED((NROWS, D), jnp.float32)]
    if with_counts:
        scratch += [pltpu.VMEM_SHARED((NROWS,), jnp.float32)] * 2
    scratch += [
        pltpu.VMEM((CH,), jnp.int32),
        pltpu.VMEM((CH,), jnp.int32),
        pltpu.VMEM((CH,), jnp.int32),
        pltpu.VMEM((CH,), jnp.int32),
        pltpu.VMEM((CH, D), jnp.float32),
        pltpu.VMEM((CH, D), jnp.float32),
        pltpu.VMEM((CH,), jnp.float32),
        pltpu.SemaphoreType.DMA,
        pltpu.SemaphoreType.DMA,
        pltpu.SemaphoreType.DMA,
        pltpu.SemaphoreType.DMA,
    ]
    if with_counts:
        scratch += [pltpu.VMEM((MMB,), jnp.float32)]

    def body(*refs):
        if with_counts:
            (table_hbm, gidx_hbm, sidx_hbm,
             acc0_out, acc1_out, hg0_out, hg1_out, hs0_out, hs1_out,
             acc_sh, hg_sh, hs_sh,
             gb0, sb0, gb1, sb1, rows0_v, rows1_v, ones_v,
             semi0, semi1, semr0, semr1, hbuf) = refs
        else:
            (table_hbm, gidx_hbm, sidx_hbm,
             acc0_out, acc1_out,
             acc_sh,
             gb0, sb0, gb1, sb1, rows0_v, rows1_v, ones_v,
             semi0, semi1, semr0, semr1) = refs
        cid = lax.axis_index("c")
        sid = lax.axis_index("s")
        wid = sid * NC + cid
        base = sid * RPT
        zv = jnp.zeros((16,), jnp.float32)

        def zrow(j, c):
            for i in range(D // 16):
                rows0_v[j, pl.ds(i * 16, 16)] = zv
            return c

        lax.fori_loop(0, CH, zrow, 0)

        def zero_slices(nrows):
            for k in range(nrows // CH):
                pltpu.sync_copy(rows0_v, acc_sh.at[pl.ds(base + k * CH, CH)])
            rem = nrows % CH
            pltpu.sync_copy(rows0_v.at[pl.ds(0, rem)],
                            acc_sh.at[pl.ds(base + nrows - rem, rem)])
            if with_counts:
                for h_sh in (hg_sh, hs_sh):
                    for k in range(nrows // CH):
                        pltpu.sync_copy(rows0_v.at[0],
                                        h_sh.at[pl.ds(base + k * CH, CH)])
                    pltpu.sync_copy(rows0_v.at[0, pl.ds(0, rem)],
                                    h_sh.at[pl.ds(base + nrows - rem, rem)])

        @pl.when(sid < 15)
        def _():
            zero_slices(RPT)

        @pl.when(sid == 15)
        def _():
            zero_slices(RPT_LAST)

        if with_counts:
            for i in range(CH // 16):
                ones_v[pl.ds(i * 16, 16)] = jnp.full((16,), 1.0, jnp.float32)
        plsc.subcore_barrier()

        base_c = NCHW * wid + jnp.minimum(wid, NEXTRA)

        def idxload(c, gb, sb, sem):
            pltpu.async_copy(gidx_hbm.at[c, 0], gb, sem)
            pltpu.async_copy(sidx_hbm.at[c, 0], sb, sem)

        def idxwait(gb, sb, sem):
            pltpu.make_async_copy(gidx_hbm.at[0, 0], gb, sem).wait()
            pltpu.make_async_copy(sidx_hbm.at[0, 0], sb, sem).wait()

        def rows_gather(gb, buf, sem):
            pltpu.async_copy(table_hbm.at[gb], buf, sem)

        def rows_wait(gb, buf, sem):
            pltpu.make_async_copy(table_hbm.at[gb], buf, sem).wait()

        def consume(gb, sb, buf):
            pltpu.sync_copy(buf, acc_sh.at[sb], add=True)
            if with_counts:
                pltpu.sync_copy(ones_v, hg_sh.at[gb], add=True)
                pltpu.sync_copy(ones_v, hs_sh.at[sb], add=True)

        idxload(base_c, gb0, sb0, semi0)
        idxwait(gb0, sb0, semi0)
        rows_gather(gb0, rows0_v, semr0)
        idxload(base_c + 1, gb1, sb1, semi1)
        idxwait(gb1, sb1, semi1)

        def pair(t, carry):
            j0 = 2 * t
            c0 = base_c + j0
            rows_gather(gb1, rows1_v, semr1)
            rows_wait(gb0, rows0_v, semr0)
            consume(gb0, sb0, rows0_v)

            @pl.when(j0 + 2 < NCHW)
            def _():
                idxload(c0 + 2, gb0, sb0, semi0)
                idxwait(gb0, sb0, semi0)
                rows_gather(gb0, rows0_v, semr0)

            rows_wait(gb1, rows1_v, semr1)
            consume(gb1, sb1, rows1_v)

            @pl.when(j0 + 3 < NCHW)
            def _():
                idxload(c0 + 3, gb1, sb1, semi1)
                idxwait(gb1, sb1, semi1)
            return carry

        lax.fori_loop(0, NCHW // 2, pair, 0)

        @pl.when(wid < NEXTRA)
        def _():
            c = base_c + NCHW
            idxload(c, gb0, sb0, semi0)
            idxwait(gb0, sb0, semi0)
            rows_gather(gb0, rows0_v, semr0)
            rows_wait(gb0, rows0_v, semr0)
            consume(gb0, sb0, rows0_v)

        plsc.subcore_barrier()

        def writeback(nrows, acc_out):
            sl = pl.ds(base, nrows)
            pltpu.sync_copy(acc_sh.at[sl], acc_out.at[sl])

        @pl.when((cid == 0) & (sid < 15))
        def _():
            writeback(RPT, acc0_out)

        @pl.when((cid == 0) & (sid == 15))
        def _():
            writeback(RPT_LAST, acc0_out)

        @pl.when((cid == 1) & (sid < 15))
        def _():
            writeback(RPT, acc1_out)

        @pl.when((cid == 1) & (sid == 15))
        def _():
            writeback(RPT_LAST, acc1_out)

        if with_counts:
            def hist_write(h_sh, h_out):
                for p in range(NROWS // MMB):
                    ps = pl.ds(p * MMB, MMB)
                    pltpu.sync_copy(h_sh.at[ps], hbuf)
                    pltpu.sync_copy(hbuf, h_out.at[ps])

            @pl.when((cid == 0) & (sid == 0))
            def _():
                hist_write(hg_sh, hg0_out)
                hist_write(hs_sh, hs0_out)

            @pl.when((cid == 1) & (sid == 0))
            def _():
                hist_write(hg_sh, hg1_out)
                hist_write(hs_sh, hs1_out)

    return pl.kernel(body, out_type=tuple(out_type), mesh=mesh,
                     scratch_types=tuple(scratch))


def _inv_cnt(c0_ref, c1_ref):
    cnt = c0_ref[...] + c1_ref[...]
    return jnp.where(cnt > 0, 1.0 / cnt, 0.0)[:, None]


def _scale_body(m0_ref, m1_ref, c0_ref, c1_ref, o_ref):
    o_ref[...] = (m0_ref[...] + m1_ref[...]) * _inv_cnt(c0_ref, c1_ref)


def _snn_body(o0_ref, o1_ref, c0_ref, c1_ref, w_ref, bias_ref, mem_ref,
              out_ref):
    agg = (o0_ref[...] + o1_ref[...]) * _inv_cnt(c0_ref, c1_ref)
    v = lax.dot_general(agg, w_ref[...], (((1,), (1,)), ((), ())),
                        preferred_element_type=jnp.float32)
    v = v + bias_ref[...] + BETA * mem_ref[...]
    out_ref[...] = (v > SPIKE_THRESHOLD).astype(jnp.float32)


def _row_spec():
    return pl.BlockSpec((MMB, D), lambda i: (i, 0))


def kernel(x, hyperedge_index, W, bias, membrane):
    nodep = hyperedge_index[0].reshape(NCHT, 1, CH)
    edgep = hyperedge_index[1].reshape(NCHT, 1, CH)

    m0, m1, hn0, hn1, he0, he1 = _make_sc_pass(True)(x, nodep, edgep)

    m_scaled = pl.pallas_call(
        _scale_body,
        out_shape=jax.ShapeDtypeStruct((NROWS, D), jnp.float32),
    )(m0, m1, he0, he1)

    o0, o1 = _make_sc_pass(False)(m_scaled, edgep, nodep)

    spike = pl.pallas_call(
        _snn_body,
        out_shape=jax.ShapeDtypeStruct((NROWS, D), jnp.float32),
    )(o0, o1, hn0, hn1, W, bias.reshape(1, D), membrane.reshape(1, D))

    return spike


# speedup vs baseline: 42.9181x; 1.1366x over previous
import functools

import jax
import jax.numpy as jnp
from jax import lax
from jax.experimental import pallas as pl
from jax.experimental.pallas import tpu as pltpu
from jax.experimental.pallas import tpu_sc as plsc

N_NODES = 10000
N_CONN = 320000
D = 128
BETA = 0.9
SPIKE_THRESHOLD = 1.0

NC = 2
NS = 16
NW = NC * NS
CH = 128
NCHT = N_CONN // CH
NCHW = (NCHT // NW) & ~1
NEXTRA = NCHT - NCHW * NW
NROWS = N_NODES
RPT = 632
RPT_LAST = NROWS - 15 * RPT
MMB = NROWS // 10


def _make_sc_pass(with_counts):
    mesh = plsc.VectorSubcoreMesh(core_axis_name="c", subcore_axis_name="s",
                                  num_cores=NC, num_subcores=NS)
    out_type = [jax.ShapeDtypeStruct((NROWS, D), jnp.float32)] * 2
    if with_counts:
        out_type += [jax.ShapeDtypeStruct((NROWS,), jnp.float32)] * 4
    scratch = [pltpu.VMEM_SHARED((NROWS, D), jnp.float32)]
    if with_counts:
        scratch += [pltpu.VMEM_SHARED((NROWS,), jnp.float32)] * 2
    scratch += [
        pltpu.VMEM((CH,), jnp.int32),
        pltpu.VMEM((CH,), jnp.int32),
        pltpu.VMEM((CH,), jnp.int32),
        pltpu.VMEM((CH,), jnp.int32),
        pltpu.VMEM((CH,), jnp.int32),
        pltpu.VMEM((CH,), jnp.int32),
        pltpu.VMEM((CH, D), jnp.float32),
        pltpu.VMEM((CH, D), jnp.float32),
        pltpu.VMEM((CH,), jnp.float32),
        pltpu.SemaphoreType.DMA,
        pltpu.SemaphoreType.DMA,
        pltpu.SemaphoreType.DMA,
        pltpu.SemaphoreType.DMA,
        pltpu.SemaphoreType.DMA,
    ]
    if with_counts:
        scratch += [pltpu.VMEM((MMB,), jnp.float32)]

    def body(*refs):
        if with_counts:
            (table_hbm, gidx_hbm, sidx_hbm,
             acc0_out, acc1_out, hg0_out, hg1_out, hs0_out, hs1_out,
             acc_sh, hg_sh, hs_sh,
             gb0, sb0, gb1, sb1, gb2, sb2, rows0_v, rows1_v, ones_v,
             semi0, semi1, semi2, semr0, semr1, hbuf) = refs
        else:
            (table_hbm, gidx_hbm, sidx_hbm,
             acc0_out, acc1_out,
             acc_sh,
             gb0, sb0, gb1, sb1, gb2, sb2, rows0_v, rows1_v, ones_v,
             semi0, semi1, semi2, semr0, semr1) = refs
        isets = ((gb0, sb0, semi0), (gb1, sb1, semi1), (gb2, sb2, semi2))
        rsets = ((rows0_v, semr0), (rows1_v, semr1))
        cid = lax.axis_index("c")
        sid = lax.axis_index("s")
        wid = sid * NC + cid
        base = sid * RPT
        zv = jnp.zeros((16,), jnp.float32)

        def zrow(j, c):
            for i in range(D // 16):
                rows0_v[j, pl.ds(i * 16, 16)] = zv
            return c

        lax.fori_loop(0, CH, zrow, 0)

        def zero_slices(nrows):
            for k in range(nrows // CH):
                pltpu.sync_copy(rows0_v, acc_sh.at[pl.ds(base + k * CH, CH)])
            rem = nrows % CH
            pltpu.sync_copy(rows0_v.at[pl.ds(0, rem)],
                            acc_sh.at[pl.ds(base + nrows - rem, rem)])
            if with_counts:
                for h_sh in (hg_sh, hs_sh):
                    for k in range(nrows // CH):
                        pltpu.sync_copy(rows0_v.at[0],
                                        h_sh.at[pl.ds(base + k * CH, CH)])
                    pltpu.sync_copy(rows0_v.at[0, pl.ds(0, rem)],
                                    h_sh.at[pl.ds(base + nrows - rem, rem)])

        @pl.when(sid < 15)
        def _():
            zero_slices(RPT)

        @pl.when(sid == 15)
        def _():
            zero_slices(RPT_LAST)

        if with_counts:
            for i in range(CH // 16):
                ones_v[pl.ds(i * 16, 16)] = jnp.full((16,), 1.0, jnp.float32)
        plsc.subcore_barrier()

        base_c = NCHW * wid + jnp.minimum(wid, NEXTRA)

        def idxload(c, gb, sb, sem):
            pltpu.async_copy(gidx_hbm.at[c, 0], gb, sem)
            pltpu.async_copy(sidx_hbm.at[c, 0], sb, sem)

        def idxwait(gb, sb, sem):
            pltpu.make_async_copy(gidx_hbm.at[0, 0], gb, sem).wait()
            pltpu.make_async_copy(sidx_hbm.at[0, 0], sb, sem).wait()

        def rows_gather(gb, buf, sem):
            pltpu.async_copy(table_hbm.at[gb], buf, sem)

        def rows_wait(gb, buf, sem):
            pltpu.make_async_copy(table_hbm.at[gb], buf, sem).wait()

        def consume(gb, sb, buf):
            pltpu.sync_copy(buf, acc_sh.at[sb], add=True)
            if with_counts:
                pltpu.sync_copy(ones_v, hg_sh.at[gb], add=True)
                pltpu.sync_copy(ones_v, hs_sh.at[sb], add=True)

        idxload(base_c, gb0, sb0, semi0)
        idxwait(gb0, sb0, semi0)
        idxload(base_c + 1, gb1, sb1, semi1)
        rows_gather(gb0, rows0_v, semr0)
        idxwait(gb1, sb1, semi1)

        def six(t, carry):
            c = base_c + 6 * t
            for k in range(6):
                gb_c, sb_c, _ = isets[k % 3]
                gb_n, sb_n, _ = isets[(k + 1) % 3]
                gb_p, sb_p, semi_p = isets[(k + 2) % 3]
                buf_c, semr_c = rsets[k % 2]
                buf_n, semr_n = rsets[(k + 1) % 2]

                @pl.when(6 * t + k + 1 < NCHW)
                def _():
                    rows_gather(gb_n, buf_n, semr_n)

                @pl.when(6 * t + k + 2 < NCHW)
                def _():
                    idxload(c + k + 2, gb_p, sb_p, semi_p)

                rows_wait(gb_c, buf_c, semr_c)
                consume(gb_c, sb_c, buf_c)

                @pl.when(6 * t + k + 2 < NCHW)
                def _():
                    idxwait(gb_p, sb_p, semi_p)
            return carry

        lax.fori_loop(0, NCHW // 6, six, 0)

        @pl.when(wid < NEXTRA)
        def _():
            c = base_c + NCHW
            idxload(c, gb0, sb0, semi0)
            idxwait(gb0, sb0, semi0)
            rows_gather(gb0, rows0_v, semr0)
            rows_wait(gb0, rows0_v, semr0)
            consume(gb0, sb0, rows0_v)

        plsc.subcore_barrier()

        def writeback(nrows, acc_out):
            sl = pl.ds(base, nrows)
            pltpu.sync_copy(acc_sh.at[sl], acc_out.at[sl])

        @pl.when((cid == 0) & (sid < 15))
        def _():
            writeback(RPT, acc0_out)

        @pl.when((cid == 0) & (sid == 15))
        def _():
            writeback(RPT_LAST, acc0_out)

        @pl.when((cid == 1) & (sid < 15))
        def _():
            writeback(RPT, acc1_out)

        @pl.when((cid == 1) & (sid == 15))
        def _():
            writeback(RPT_LAST, acc1_out)

        if with_counts:
            def hist_write(h_sh, h_out):
                for p in range(NROWS // MMB):
                    ps = pl.ds(p * MMB, MMB)
                    pltpu.sync_copy(h_sh.at[ps], hbuf)
                    pltpu.sync_copy(hbuf, h_out.at[ps])

            @pl.when((cid == 0) & (sid == 0))
            def _():
                hist_write(hg_sh, hg0_out)
                hist_write(hs_sh, hs0_out)

            @pl.when((cid == 1) & (sid == 0))
            def _():
                hist_write(hg_sh, hg1_out)
                hist_write(hs_sh, hs1_out)

    return pl.kernel(body, out_type=tuple(out_type), mesh=mesh,
                     scratch_types=tuple(scratch))


def _inv_cnt(c0_ref, c1_ref):
    cnt = c0_ref[...] + c1_ref[...]
    return jnp.where(cnt > 0, 1.0 / cnt, 0.0)[:, None]


def _scale_body(m0_ref, m1_ref, c0_ref, c1_ref, o_ref):
    o_ref[...] = (m0_ref[...] + m1_ref[...]) * _inv_cnt(c0_ref, c1_ref)


def _snn_body(o0_ref, o1_ref, c0_ref, c1_ref, w_ref, bias_ref, mem_ref,
              out_ref):
    agg = (o0_ref[...] + o1_ref[...]) * _inv_cnt(c0_ref, c1_ref)
    v = lax.dot_general(agg, w_ref[...], (((1,), (1,)), ((), ())),
                        preferred_element_type=jnp.float32)
    v = v + bias_ref[...] + BETA * mem_ref[...]
    out_ref[...] = (v > SPIKE_THRESHOLD).astype(jnp.float32)


def _row_spec():
    return pl.BlockSpec((MMB, D), lambda i: (i, 0))


def kernel(x, hyperedge_index, W, bias, membrane):
    nodep = hyperedge_index[0].reshape(NCHT, 1, CH)
    edgep = hyperedge_index[1].reshape(NCHT, 1, CH)

    m0, m1, hn0, hn1, he0, he1 = _make_sc_pass(True)(x, nodep, edgep)

    m_scaled = pl.pallas_call(
        _scale_body,
        out_shape=jax.ShapeDtypeStruct((NROWS, D), jnp.float32),
    )(m0, m1, he0, he1)

    o0, o1 = _make_sc_pass(False)(m_scaled, edgep, nodep)

    spike = pl.pallas_call(
        _snn_body,
        out_shape=jax.ShapeDtypeStruct((NROWS, D), jnp.float32),
    )(o0, o1, hn0, hn1, W, bias.reshape(1, D), membrane.reshape(1, D))

    return spike


# speedup vs baseline: 43.4633x; 1.0127x over previous
import functools

import jax
import jax.numpy as jnp
from jax import lax
from jax.experimental import pallas as pl
from jax.experimental.pallas import tpu as pltpu
from jax.experimental.pallas import tpu_sc as plsc

N_NODES = 10000
N_CONN = 320000
D = 128
BETA = 0.9
SPIKE_THRESHOLD = 1.0

NC = 2
NS = 16
NW = NC * NS
CH = 128
NCHT = N_CONN // CH
NCHW = (NCHT // NW) & ~1
NEXTRA = NCHT - NCHW * NW
NROWS = N_NODES
RPT = 632
RPT_LAST = NROWS - 15 * RPT
MMB = NROWS // 10


def _make_sc_pass(with_counts):
    mesh = plsc.VectorSubcoreMesh(core_axis_name="c", subcore_axis_name="s",
                                  num_cores=NC, num_subcores=NS)
    out_type = [jax.ShapeDtypeStruct((NROWS, D), jnp.float32)] * 2
    if with_counts:
        out_type += [jax.ShapeDtypeStruct((NROWS,), jnp.float32)] * 4
    scratch = [pltpu.VMEM_SHARED((NROWS, D), jnp.float32)]
    if with_counts:
        scratch += [pltpu.VMEM_SHARED((NROWS,), jnp.float32)] * 2
    scratch += [
        pltpu.VMEM((CH,), jnp.int32),
        pltpu.VMEM((CH,), jnp.int32),
        pltpu.VMEM((CH,), jnp.int32),
        pltpu.VMEM((CH,), jnp.int32),
        pltpu.VMEM((CH,), jnp.int32),
        pltpu.VMEM((CH,), jnp.int32),
        pltpu.VMEM((CH, D), jnp.float32),
        pltpu.VMEM((CH, D), jnp.float32),
        pltpu.VMEM((CH,), jnp.float32),
        pltpu.SemaphoreType.DMA,
        pltpu.SemaphoreType.DMA,
        pltpu.SemaphoreType.DMA,
        pltpu.SemaphoreType.DMA,
        pltpu.SemaphoreType.DMA,
        pltpu.SemaphoreType.DMA,
        pltpu.SemaphoreType.DMA,
    ]
    if with_counts:
        scratch += [pltpu.VMEM((MMB,), jnp.float32)]

    def body(*refs):
        if with_counts:
            (table_hbm, gidx_hbm, sidx_hbm,
             acc0_out, acc1_out, hg0_out, hg1_out, hs0_out, hs1_out,
             acc_sh, hg_sh, hs_sh,
             gb0, sb0, gb1, sb1, gb2, sb2, rows0_v, rows1_v, ones_v,
             semi0, semi1, semi2, semr0, semr1, sems0, sems1, hbuf) = refs
        else:
            (table_hbm, gidx_hbm, sidx_hbm,
             acc0_out, acc1_out,
             acc_sh,
             gb0, sb0, gb1, sb1, gb2, sb2, rows0_v, rows1_v, ones_v,
             semi0, semi1, semi2, semr0, semr1, sems0, sems1) = refs
        isets = ((gb0, sb0, semi0), (gb1, sb1, semi1), (gb2, sb2, semi2))
        rsets = ((rows0_v, semr0, sems0), (rows1_v, semr1, sems1))
        cid = lax.axis_index("c")
        sid = lax.axis_index("s")
        wid = sid * NC + cid
        base = sid * RPT
        zv = jnp.zeros((16,), jnp.float32)

        def zrow(j, c):
            for i in range(D // 16):
                rows0_v[j, pl.ds(i * 16, 16)] = zv
            return c

        lax.fori_loop(0, CH, zrow, 0)

        def zero_slices(nrows):
            for k in range(nrows // CH):
                pltpu.sync_copy(rows0_v, acc_sh.at[pl.ds(base + k * CH, CH)])
            rem = nrows % CH
            pltpu.sync_copy(rows0_v.at[pl.ds(0, rem)],
                            acc_sh.at[pl.ds(base + nrows - rem, rem)])
            if with_counts:
                for h_sh in (hg_sh, hs_sh):
                    for k in range(nrows // CH):
                        pltpu.sync_copy(rows0_v.at[0],
                                        h_sh.at[pl.ds(base + k * CH, CH)])
                    pltpu.sync_copy(rows0_v.at[0, pl.ds(0, rem)],
                                    h_sh.at[pl.ds(base + nrows - rem, rem)])

        @pl.when(sid < 15)
        def _():
            zero_slices(RPT)

        @pl.when(sid == 15)
        def _():
            zero_slices(RPT_LAST)

        if with_counts:
            for i in range(CH // 16):
                ones_v[pl.ds(i * 16, 16)] = jnp.full((16,), 1.0, jnp.float32)
        plsc.subcore_barrier()

        base_c = NCHW * wid + jnp.minimum(wid, NEXTRA)

        def idxload(c, gb, sb, sem):
            pltpu.async_copy(gidx_hbm.at[c, 0], gb, sem)
            pltpu.async_copy(sidx_hbm.at[c, 0], sb, sem)

        def idxwait(gb, sb, sem):
            pltpu.make_async_copy(gidx_hbm.at[0, 0], gb, sem).wait()
            pltpu.make_async_copy(sidx_hbm.at[0, 0], sb, sem).wait()

        def rows_gather(gb, buf, sem):
            pltpu.async_copy(table_hbm.at[gb], buf, sem)

        def rows_wait(gb, buf, sem):
            pltpu.make_async_copy(table_hbm.at[gb], buf, sem).wait()

        def consume(gb, sb, buf):
            pltpu.sync_copy(buf, acc_sh.at[sb], add=True)
            if with_counts:
                pltpu.sync_copy(ones_v, hg_sh.at[gb], add=True)
                pltpu.sync_copy(ones_v, hs_sh.at[sb], add=True)

        def consume_async(gb, sb, buf, sem):
            pltpu.async_copy(buf, acc_sh.at[sb], sem, add=True)
            if with_counts:
                pltpu.async_copy(ones_v, hg_sh.at[gb], sem, add=True)
                pltpu.async_copy(ones_v, hs_sh.at[sb], sem, add=True)

        def consume_wait(sem):
            pltpu.make_async_copy(rows0_v, acc_sh.at[sb0], sem).wait()
            if with_counts:
                pltpu.make_async_copy(ones_v, hg_sh.at[gb0], sem).wait()
                pltpu.make_async_copy(ones_v, hs_sh.at[sb0], sem).wait()

        idxload(base_c, gb0, sb0, semi0)
        idxwait(gb0, sb0, semi0)
        idxload(base_c + 1, gb1, sb1, semi1)
        rows_gather(gb0, rows0_v, semr0)
        idxwait(gb1, sb1, semi1)

        def six(t, carry):
            c = base_c + 6 * t
            for k in range(6):
                gb_c, sb_c, _ = isets[k % 3]
                gb_n, sb_n, _ = isets[(k + 1) % 3]
                gb_p, sb_p, semi_p = isets[(k + 2) % 3]
                buf_c, semr_c, sems_c = rsets[k % 2]
                buf_n, semr_n, sems_n = rsets[(k + 1) % 2]

                j = 6 * t + k
                gcond = j + 1 < NCHW if k < 5 else (6 * t + 6 < NCHW)
                if k == 0:
                    gcond = gcond & (t > 0)

                    @pl.when(t == 0)
                    def _():
                        rows_gather(gb_n, buf_n, semr_n)

                @pl.when(gcond)
                def _():
                    consume_wait(sems_n)
                    rows_gather(gb_n, buf_n, semr_n)

                @pl.when(j + 2 < NCHW)
                def _():
                    idxload(c + k + 2, gb_p, sb_p, semi_p)

                rows_wait(gb_c, buf_c, semr_c)
                consume_async(gb_c, sb_c, buf_c, sems_c)

                @pl.when(j + 2 < NCHW)
                def _():
                    idxwait(gb_p, sb_p, semi_p)
            return carry

        lax.fori_loop(0, NCHW // 6, six, 0)
        consume_wait(sems0)
        consume_wait(sems1)

        @pl.when(wid < NEXTRA)
        def _():
            c = base_c + NCHW
            idxload(c, gb0, sb0, semi0)
            idxwait(gb0, sb0, semi0)
            rows_gather(gb0, rows0_v, semr0)
            rows_wait(gb0, rows0_v, semr0)
            consume(gb0, sb0, rows0_v)

        plsc.subcore_barrier()

        def writeback(nrows, acc_out):
            sl = pl.ds(base, nrows)
            pltpu.sync_copy(acc_sh.at[sl], acc_out.at[sl])

        @pl.when((cid == 0) & (sid < 15))
        def _():
            writeback(RPT, acc0_out)

        @pl.when((cid == 0) & (sid == 15))
        def _():
            writeback(RPT_LAST, acc0_out)

        @pl.when((cid == 1) & (sid < 15))
        def _():
            writeback(RPT, acc1_out)

        @pl.when((cid == 1) & (sid == 15))
        def _():
            writeback(RPT_LAST, acc1_out)

        if with_counts:
            def hist_write(h_sh, h_out):
                for p in range(NROWS // MMB):
                    ps = pl.ds(p * MMB, MMB)
                    pltpu.sync_copy(h_sh.at[ps], hbuf)
                    pltpu.sync_copy(hbuf, h_out.at[ps])

            @pl.when((cid == 0) & (sid == 0))
            def _():
                hist_write(hg_sh, hg0_out)
                hist_write(hs_sh, hs0_out)

            @pl.when((cid == 1) & (sid == 0))
            def _():
                hist_write(hg_sh, hg1_out)
                hist_write(hs_sh, hs1_out)

    return pl.kernel(body, out_type=tuple(out_type), mesh=mesh,
                     scratch_types=tuple(scratch))


def _inv_cnt(c0_ref, c1_ref):
    cnt = c0_ref[...] + c1_ref[...]
    return jnp.where(cnt > 0, 1.0 / cnt, 0.0)[:, None]


def _scale_body(m0_ref, m1_ref, c0_ref, c1_ref, o_ref):
    o_ref[...] = (m0_ref[...] + m1_ref[...]) * _inv_cnt(c0_ref, c1_ref)


def _snn_body(o0_ref, o1_ref, c0_ref, c1_ref, w_ref, bias_ref, mem_ref,
              out_ref):
    agg = (o0_ref[...] + o1_ref[...]) * _inv_cnt(c0_ref, c1_ref)
    v = lax.dot_general(agg, w_ref[...], (((1,), (1,)), ((), ())),
                        preferred_element_type=jnp.float32)
    v = v + bias_ref[...] + BETA * mem_ref[...]
    out_ref[...] = (v > SPIKE_THRESHOLD).astype(jnp.float32)


def _row_spec():
    return pl.BlockSpec((MMB, D), lambda i: (i, 0))


def kernel(x, hyperedge_index, W, bias, membrane):
    nodep = hyperedge_index[0].reshape(NCHT, 1, CH)
    edgep = hyperedge_index[1].reshape(NCHT, 1, CH)

    m0, m1, hn0, hn1, he0, he1 = _make_sc_pass(True)(x, nodep, edgep)

    m_scaled = pl.pallas_call(
        _scale_body,
        out_shape=jax.ShapeDtypeStruct((NROWS, D), jnp.float32),
    )(m0, m1, he0, he1)

    o0, o1 = _make_sc_pass(False)(m_scaled, edgep, nodep)

    spike = pl.pallas_call(
        _snn_body,
        out_shape=jax.ShapeDtypeStruct((NROWS, D), jnp.float32),
    )(o0, o1, hn0, hn1, W, bias.reshape(1, D), membrane.reshape(1, D))

    return spike
